# baseline (speedup 1.0000x reference)
import functools
import math

import jax
import jax.numpy as jnp
from jax import lax
from jax.experimental import pallas as pl
from jax.experimental.pallas import tpu as pltpu
from jax.experimental.pallas import tpu_sc as plsc

N = 10000
E = 160000
D = 128
DQK = 32
DATTR = 16
NB = 8
RMAX = 5.0
PPOW = 6.0
NLAYERS = 2

NC = 2
NS = 16
NW = NC * NS
NP = 10240
EP = 163840
ET = EP // NW
BATCH = 128
NBATCH = ET // BATCH
NGRP = ET // 16

_NEG_INF = float("-inf")


def _silu(x):
    return x * jax.nn.sigmoid(x)


def _k_pre_body(ev_ref, freq_ref, es_ref, ea_ref):
    ev = ev_ref[...]
    r2 = jnp.sum(ev * ev, axis=1, keepdims=True)
    r = jnp.sqrt(r2)
    freq = freq_ref[...]
    x = r / RMAX
    p = PPOW
    cut = (1.0
           - ((p + 1.0) * (p + 2.0) / 2.0) * x ** 6
           + p * (p + 2.0) * x ** 7
           - (p * (p + 1.0) / 2.0) * x ** 8)
    cut = cut * (x < 1.0).astype(jnp.float32)
    es = (2.0 / RMAX) * jnp.sin(freq * r / RMAX) / r * cut
    es_ref[...] = es
    u = ev / r
    s3 = math.sqrt(3.0)
    ea_ref[...] = jnp.concatenate(
        [jnp.ones_like(r), s3 * u[:, 1:2], s3 * u[:, 2:3], s3 * u[:, 0:1]],
        axis=1)


def _edge_pre(ev_p, freq):
    BE = 2048
    grid = (EP // BE,)
    return pl.pallas_call(
        _k_pre_body,
        grid=grid,
        in_specs=[
            pl.BlockSpec((BE, 3), lambda i: (i, 0)),
            pl.BlockSpec((1, NB), lambda i: (0, 0)),
        ],
        out_specs=[
            pl.BlockSpec((BE, NB), lambda i: (i, 0)),
            pl.BlockSpec((BE, 4), lambda i: (i, 0)),
        ],
        out_shape=[
            jax.ShapeDtypeStruct((EP, NB), jnp.float32),
            jax.ShapeDtypeStruct((EP, 4), jnp.float32),
        ],
    )(ev_p, freq.reshape(1, NB))


def _k_tables_body(x_ref, na_ref, wq_ref, wk1_ref, wv1_ref, ws1_ref, ws2_ref,
                   qk_ref, vt_ref, sc_ref):
    x = x_ref[...]
    q = jnp.dot(x, wq_ref[...], preferred_element_type=jnp.float32)
    k = jnp.dot(x, wk1_ref[...], preferred_element_type=jnp.float32)
    z = jnp.zeros_like(x[:, :64])
    qk_ref[...] = jnp.concatenate([q, k, z], axis=1)
    vt_ref[...] = jnp.dot(x, wv1_ref[...], preferred_element_type=jnp.float32)
    sc_ref[...] = (jnp.dot(x, ws1_ref[...], preferred_element_type=jnp.float32)
                   * jnp.dot(na_ref[...], ws2_ref[...],
                             preferred_element_type=jnp.float32))


def _tables(x_p, na_p, wq, wk1, wv1, ws1, ws2):
    BN = 1024
    grid = (NP // BN,)
    return pl.pallas_call(
        _k_tables_body,
        grid=grid,
        in_specs=[
            pl.BlockSpec((BN, D), lambda i: (i, 0)),
            pl.BlockSpec((BN, DATTR), lambda i: (i, 0)),
            pl.BlockSpec((D, DQK), lambda i: (0, 0)),
            pl.BlockSpec((D, DQK), lambda i: (0, 0)),
            pl.BlockSpec((D, D), lambda i: (0, 0)),
            pl.BlockSpec((D, D), lambda i: (0, 0)),
            pl.BlockSpec((DATTR, D), lambda i: (0, 0)),
        ],
        out_specs=[
            pl.BlockSpec((BN, D), lambda i: (i, 0)),
            pl.BlockSpec((BN, D), lambda i: (i, 0)),
            pl.BlockSpec((BN, D), lambda i: (i, 0)),
        ],
        out_shape=[
            jax.ShapeDtypeStruct((NP, D), jnp.float32),
            jax.ShapeDtypeStruct((NP, D), jnp.float32),
            jax.ShapeDtypeStruct((NP, D), jnp.float32),
        ],
    )(x_p, na_p, wq, wk1, wv1, ws1, ws2)


def _k_logit_body(qkg_ref, es_ref, ea_ref, f1_ref, f2_ref, f3_ref,
                  wk2_ref, out_ref):
    h = _silu(jnp.dot(es_ref[...], f1_ref[...],
                      preferred_element_type=jnp.float32))
    h = _silu(jnp.dot(h, f2_ref[...], preferred_element_type=jnp.float32))
    wk = jnp.dot(h, f3_ref[...], preferred_element_type=jnp.float32)
    ck = wk * jnp.dot(ea_ref[...], wk2_ref[...],
                      preferred_element_type=jnp.float32)
    qkg = qkg_ref[...]
    t = qkg[:, 0:DQK] * qkg[:, DQK:2 * DQK] * ck
    out_ref[...] = jnp.sum(t, axis=1, keepdims=True) * (1.0 / math.sqrt(DQK))


def _logits(qkg, es, ea, f1, f2, f3, wk2):
    BE = 2048
    grid = (EP // BE,)
    return pl.pallas_call(
        _k_logit_body,
        grid=grid,
        in_specs=[
            pl.BlockSpec((BE, D), lambda i: (i, 0)),
            pl.BlockSpec((BE, NB), lambda i: (i, 0)),
            pl.BlockSpec((BE, 4), lambda i: (i, 0)),
            pl.BlockSpec((NB, 8), lambda i: (0, 0)),
            pl.BlockSpec((8, 8), lambda i: (0, 0)),
            pl.BlockSpec((8, DQK), lambda i: (0, 0)),
            pl.BlockSpec((4, DQK), lambda i: (0, 0)),
        ],
        out_specs=pl.BlockSpec((BE, 1), lambda i: (i, 0)),
        out_shape=jax.ShapeDtypeStruct((EP, 1), jnp.float32),
    )(qkg, es, ea, f1, f2, f3, wk2)


def _k_reduce_body(part_ref, out_ref, *, op):
    t = jnp.transpose(part_ref[...])
    if op == "max":
        r = jnp.max(t, axis=1, keepdims=True)
        r = jnp.where(jnp.isfinite(r), r, 0.0)
    else:
        r = jnp.sum(t, axis=1, keepdims=True)
    out_ref[...] = r


def _reduce_parts(part, op):
    BN = 512
    grid = (NP // BN,)
    return pl.pallas_call(
        functools.partial(_k_reduce_body, op=op),
        grid=grid,
        in_specs=[pl.BlockSpec((NW, BN), lambda i: (0, i))],
        out_specs=pl.BlockSpec((BN, 1), lambda i: (i, 0)),
        out_shape=jax.ShapeDtypeStruct((NP, 1), jnp.float32),
    )(part)


def _k_wcv_body(es_ref, ea_ref, ex_ref, f1_ref, f2_ref, f3_ref, wv2_ref,
                out_ref):
    h = _silu(jnp.dot(es_ref[...], f1_ref[...],
                      preferred_element_type=jnp.float32))
    h = _silu(jnp.dot(h, f2_ref[...], preferred_element_type=jnp.float32))
    wv = jnp.dot(h, f3_ref[...], preferred_element_type=jnp.float32)
    out_ref[...] = ex_ref[...] * wv * jnp.dot(
        ea_ref[...], wv2_ref[...], preferred_element_type=jnp.float32)


def _wcv(es, ea, ex, f1, f2, f3, wv2):
    BE = 2048
    grid = (EP // BE,)
    return pl.pallas_call(
        _k_wcv_body,
        grid=grid,
        in_specs=[
            pl.BlockSpec((BE, NB), lambda i: (i, 0)),
            pl.BlockSpec((BE, 4), lambda i: (i, 0)),
            pl.BlockSpec((BE, 1), lambda i: (i, 0)),
            pl.BlockSpec((NB, 8), lambda i: (0, 0)),
            pl.BlockSpec((8, 8), lambda i: (0, 0)),
            pl.BlockSpec((8, D), lambda i: (0, 0)),
            pl.BlockSpec((4, D), lambda i: (0, 0)),
        ],
        out_specs=pl.BlockSpec((BE, D), lambda i: (i, 0)),
        out_shape=jax.ShapeDtypeStruct((EP, D), jnp.float32),
    )(es, ea, ex, f1, f2, f3, wv2)


def _k_combine_body(agg_ref, den_ref, sc_ref, out_ref):
    a = agg_ref[0] + agg_ref[1]
    out_ref[...] = sc_ref[...] + a / (den_ref[...] + 1e-9)


def _combine(agg, den, sc):
    BN = 1024
    grid = (NP // BN,)
    return pl.pallas_call(
        _k_combine_body,
        grid=grid,
        in_specs=[
            pl.BlockSpec((2, BN, D), lambda i: (0, i, 0)),
            pl.BlockSpec((BN, 1), lambda i: (i, 0)),
            pl.BlockSpec((BN, D), lambda i: (i, 0)),
        ],
        out_specs=pl.BlockSpec((BN, D), lambda i: (i, 0)),
        out_shape=jax.ShapeDtypeStruct((NP, D), jnp.float32),
    )(agg, den, sc)


@functools.cache
def _mesh():
    return plsc.VectorSubcoreMesh(core_axis_name="c", subcore_axis_name="s",
                                  num_cores=NC, num_subcores=NS)


def _wid():
    return lax.axis_index("s") * NC + lax.axis_index("c")


NCHAIN = 4


def _seg_scan_rmw(keys, vals, kbuf, cbuf, arr, op):
    ident = _NEG_INF if op == "max" else 0.0
    sk, sv = plsc.sort_key_val(keys, vals)
    kbuf[pl.ds(16, 16)] = sk
    nxt = kbuf[pl.ds(17, 16)]
    last = sk != nxt
    c = sv
    for s in (1, 2, 4, 8):
        cbuf[pl.ds(16, 16)] = c
        ks = kbuf[pl.ds(16 - s, 16)]
        cs = cbuf[pl.ds(16 - s, 16)]
        eq = sk == ks
        contrib = jnp.where(eq, cs, jnp.full((16,), ident, jnp.float32))
        if op == "max":
            c = jnp.maximum(c, contrib)
        else:
            c = c + contrib
    cur = plsc.load_gather(arr, [sk], mask=last)
    if op == "max":
        new = jnp.maximum(cur, c)
    else:
        new = cur + c
    plsc.store_scatter(arr, [sk], new, mask=last)


def _fill(ref, n, value, dtype):
    v = jnp.full((16,), value, dtype)

    def body(i, _):
        ref[pl.ds(i * 16, 16)] = v
        return 0

    lax.fori_loop(0, n // 16, body, 0)


def _sc_gather_qk(qkt, dst_p, src_p):
    def body(qkt_hbm, dst_hbm, src_hbm, qkg_hbm,
             idx_d, idx_s, rows_q, rows_k, sem_q, sem_k):
        base = _wid() * ET

        def step(g, _):
            off = base + g * BATCH
            pltpu.sync_copy(dst_hbm.at[pl.ds(off, BATCH)], idx_d)
            pltpu.sync_copy(src_hbm.at[pl.ds(off, BATCH)], idx_s)
            cq = pltpu.async_copy(qkt_hbm.at[idx_d], rows_q, sem_q)
            ck = pltpu.async_copy(qkt_hbm.at[idx_s], rows_k, sem_k)
            cq.wait()
            ck.wait()

            def merge(r, _):
                rows_q[r, pl.ds(DQK, 16)] = rows_k[r, pl.ds(DQK, 16)]
                rows_q[r, pl.ds(DQK + 16, 16)] = rows_k[r, pl.ds(DQK + 16, 16)]
                return 0

            lax.fori_loop(0, BATCH, merge, 0)
            pltpu.sync_copy(rows_q, qkg_hbm.at[pl.ds(off, BATCH), :])
            return 0

        lax.fori_loop(0, NBATCH, step, 0)

    fn = pl.kernel(
        body,
        out_type=jax.ShapeDtypeStruct((EP, D), jnp.float32),
        mesh=_mesh(),
        compiler_params=pltpu.CompilerParams(needs_layout_passes=False),
        scratch_types=[
            pltpu.VMEM((BATCH,), jnp.int32),
            pltpu.VMEM((BATCH,), jnp.int32),
            pltpu.VMEM((BATCH, D), jnp.float32),
            pltpu.VMEM((BATCH, D), jnp.float32),
            pltpu.SemaphoreType.DMA,
            pltpu.SemaphoreType.DMA,
        ],
    )
    return fn(qkt, dst_p, src_p)


def _sc_segmax(logit, dst_p):
    def body(logit_hbm, dst_hbm, mpart_hbm,
             mv0, mv1, mv2, mv3, logit_v, dst_v,
             kb0, kb1, kb2, kb3, cb0, cb1, cb2, cb3):
        mvs = (mv0, mv1, mv2, mv3)
        kbs = (kb0, kb1, kb2, kb3)
        cbs = (cb0, cb1, cb2, cb3)
        w = _wid()
        base = w * ET
        pltpu.sync_copy(logit_hbm.at[pl.ds(base, ET)], logit_v)
        pltpu.sync_copy(dst_hbm.at[pl.ds(base, ET)], dst_v)
        ninf = jnp.full((16,), _NEG_INF, jnp.float32)

        def ifill(i, _):
            for c in range(NCHAIN):
                mvs[c][pl.ds(i * 16, 16)] = ninf
            return 0

        lax.fori_loop(0, NP // 16, ifill, 0)
        for c in range(NCHAIN):
            kbs[c][pl.ds(0, 16)] = jnp.full((16,), -1, jnp.int32)
            kbs[c][pl.ds(32, 16)] = jnp.full((16,), -2, jnp.int32)
            cbs[c][pl.ds(0, 16)] = ninf

        def step(go, _):
            for c in range(NCHAIN):
                off = go * (16 * NCHAIN) + c * 16
                keys = dst_v[pl.ds(off, 16)]
                vals = logit_v[pl.ds(off, 16)]
                _seg_scan_rmw(keys, vals, kbs[c], cbs[c], mvs[c], "max")
            return 0

        lax.fori_loop(0, NGRP // NCHAIN, step, 0)

        def mstep(i, _):
            sl = pl.ds(i * 16, 16)
            m01 = jnp.maximum(mv0[sl], mv1[sl])
            m23 = jnp.maximum(mv2[sl], mv3[sl])
            mv0[sl] = jnp.maximum(m01, m23)
            return 0

        lax.fori_loop(0, NP // 16, mstep, 0)
        pltpu.sync_copy(mv0, mpart_hbm.at[w])

    fn = pl.kernel(
        body,
        out_type=jax.ShapeDtypeStruct((NW, NP), jnp.float32),
        mesh=_mesh(),
        compiler_params=pltpu.CompilerParams(needs_layout_passes=False),
        scratch_types=(
            [pltpu.VMEM((NP,), jnp.float32)] * NCHAIN
            + [pltpu.VMEM((ET,), jnp.float32), pltpu.VMEM((ET,), jnp.int32)]
            + [pltpu.VMEM((48,), jnp.int32)] * NCHAIN
            + [pltpu.VMEM((48,), jnp.float32)] * NCHAIN
        ),
    )
    return fn(logit, dst_p)


def _sc_exden(logit, dst_p, m):
    def body(logit_hbm, dst_hbm, m_hbm, ex_hbm, dpart_hbm,
             mv, dv0, dv1, dv2, dv3, logit_v, dst_v, ex_v,
             kb0, kb1, kb2, kb3, cb0, cb1, cb2, cb3):
        dvs = (dv0, dv1, dv2, dv3)
        kbs = (kb0, kb1, kb2, kb3)
        cbs = (cb0, cb1, cb2, cb3)
        w = _wid()
        base = w * ET
        pltpu.sync_copy(logit_hbm.at[pl.ds(base, ET)], logit_v)
        pltpu.sync_copy(dst_hbm.at[pl.ds(base, ET)], dst_v)
        pltpu.sync_copy(m_hbm, mv)
        zero = jnp.zeros((16,), jnp.float32)

        def ifill(i, _):
            for c in range(NCHAIN):
                dvs[c][pl.ds(i * 16, 16)] = zero
            return 0

        lax.fori_loop(0, NP // 16, ifill, 0)
        for c in range(NCHAIN):
            kbs[c][pl.ds(0, 16)] = jnp.full((16,), -1, jnp.int32)
            kbs[c][pl.ds(32, 16)] = jnp.full((16,), -2, jnp.int32)
            cbs[c][pl.ds(0, 16)] = zero

        def step(go, _):
            for c in range(NCHAIN):
                off = go * (16 * NCHAIN) + c * 16
                keys = dst_v[pl.ds(off, 16)]
                lg = logit_v[pl.ds(off, 16)]
                md = plsc.load_gather(mv, [keys])
                ex = jnp.exp(lg - md)
                ex_v[pl.ds(off, 16)] = ex
                _seg_scan_rmw(keys, ex, kbs[c], cbs[c], dvs[c], "sum")
            return 0

        lax.fori_loop(0, NGRP // NCHAIN, step, 0)

        def mstep(i, _):
            sl = pl.ds(i * 16, 16)
            d01 = dv0[sl] + dv1[sl]
            d23 = dv2[sl] + dv3[sl]
            dv0[sl] = d01 + d23
            return 0

        lax.fori_loop(0, NP // 16, mstep, 0)
        pltpu.sync_copy(ex_v, ex_hbm.at[pl.ds(base, ET)])
        pltpu.sync_copy(dv0, dpart_hbm.at[w])

    fn = pl.kernel(
        body,
        out_type=(
            jax.ShapeDtypeStruct((EP,), jnp.float32),
            jax.ShapeDtypeStruct((NW, NP), jnp.float32),
        ),
        mesh=_mesh(),
        compiler_params=pltpu.CompilerParams(needs_layout_passes=False),
        scratch_types=(
            [pltpu.VMEM((NP,), jnp.float32)] * (NCHAIN + 1)
            + [pltpu.VMEM((ET,), jnp.float32), pltpu.VMEM((ET,), jnp.int32),
               pltpu.VMEM((ET,), jnp.float32)]
            + [pltpu.VMEM((48,), jnp.int32)] * NCHAIN
            + [pltpu.VMEM((48,), jnp.float32)] * NCHAIN
        ),
    )
    return fn(logit, dst_p, m)


def _sc_agg(vt, wcv, src_p, dst_p):
    ROWS_PER_TILE = NP // NS
    ZCH = 64

    def body(vt_hbm, wcv_hbm, src_hbm, dst_hbm, agg_hbm,
             idx_s, idx_d, vrows, wrows, zbuf, shared_agg, sem_v):
        c = lax.axis_index("c")
        s = lax.axis_index("s")
        base = _wid() * ET

        zv = jnp.zeros((16,), jnp.float32)

        def zfill(r, _):
            for cc in range(D // 16):
                zbuf[r, pl.ds(cc * 16, 16)] = zv
            return 0

        lax.fori_loop(0, ZCH, zfill, 0)
        r0 = s * ROWS_PER_TILE

        def zstep(i, _):
            pltpu.sync_copy(zbuf, shared_agg.at[pl.ds(r0 + i * ZCH, ZCH), :])
            return 0

        lax.fori_loop(0, ROWS_PER_TILE // ZCH, zstep, 0)
        plsc.subcore_barrier()

        def step(g, _):
            off = base + g * BATCH
            pltpu.sync_copy(src_hbm.at[pl.ds(off, BATCH)], idx_s)
            cv = pltpu.async_copy(vt_hbm.at[idx_s], vrows, sem_v)
            pltpu.sync_copy(wcv_hbm.at[pl.ds(off, BATCH), :], wrows)
            pltpu.sync_copy(dst_hbm.at[pl.ds(off, BATCH)], idx_d)
            cv.wait()

            def mul_row(r, _):
                for cc in range(D // 16):
                    sl = pl.ds(cc * 16, 16)
                    vrows[r, sl] = vrows[r, sl] * wrows[r, sl]
                return 0

            lax.fori_loop(0, BATCH, mul_row, 0)
            pltpu.sync_copy(vrows, shared_agg.at[idx_d], add=True)
            return 0

        lax.fori_loop(0, NBATCH, step, 0)
        plsc.subcore_barrier()
        pltpu.sync_copy(shared_agg.at[pl.ds(r0, ROWS_PER_TILE), :],
                        agg_hbm.at[c, pl.ds(r0, ROWS_PER_TILE), :])

    fn = pl.kernel(
        body,
        out_type=jax.ShapeDtypeStruct((NC, NP, D), jnp.float32),
        mesh=_mesh(),
        compiler_params=pltpu.CompilerParams(needs_layout_passes=False),
        scratch_types=[
            pltpu.VMEM((BATCH,), jnp.int32),
            pltpu.VMEM((BATCH,), jnp.int32),
            pltpu.VMEM((BATCH, D), jnp.float32),
            pltpu.VMEM((BATCH, D), jnp.float32),
            pltpu.VMEM((ZCH, D), jnp.float32),
            pltpu.VMEM_SHARED((NP, D), jnp.float32),
            pltpu.SemaphoreType.DMA,
        ],
    )
    return fn(vt, wcv, src_p, dst_p)


def kernel(node_feature, node_attr, edge_src, edge_dst, edge_vec, bessel_freq,
           Wq, Wk1, Wk2, Wv1, Wv2, fck_w1, fck_w2, fck_w3,
           fcv_w1, fcv_w2, fcv_w3, Wsc1, Wsc2):
    f32 = jnp.float32
    x = jnp.pad(node_feature, ((0, NP - N), (0, 0)))
    na = jnp.pad(node_attr, ((0, NP - N), (0, 0)))
    pad_e = EP - E
    ev_p = jnp.concatenate(
        [edge_vec,
         jnp.broadcast_to(jnp.array([1.0, 0.0, 0.0], f32), (pad_e, 3))])
    src_p = jnp.concatenate([edge_src, jnp.full((pad_e,), N, jnp.int32)])
    dst_p = jnp.concatenate([edge_dst, jnp.full((pad_e,), N, jnp.int32)])

    es, ea = _edge_pre(ev_p, bessel_freq)

    for i in range(NLAYERS):
        qkt, vt, sc = _tables(x, na, Wq[i], Wk1[i], Wv1[i], Wsc1[i],
                              Wsc2[i])
        qkg = _sc_gather_qk(qkt, dst_p, src_p)
        logit2d = _logits(qkg, es, ea, fck_w1[i], fck_w2[i], fck_w3[i],
                          Wk2[i])
        logit = logit2d.reshape(EP)
        mpart = _sc_segmax(logit, dst_p)
        m = _reduce_parts(mpart, "max")
        ex, dpart = _sc_exden(logit, dst_p, m.reshape(NP))
        den = _reduce_parts(dpart, "sum")
        wcv = _wcv(es, ea, ex.reshape(EP, 1), fcv_w1[i], fcv_w2[i], fcv_w3[i],
                   Wv2[i])
        agg = _sc_agg(vt, wcv, src_p, dst_p)
        x = _combine(agg, den, sc)

    return x[:N]


# speedup vs baseline: 1.1324x; 1.1324x over previous
import functools
import math

import jax
import jax.numpy as jnp
from jax import lax
from jax.experimental import pallas as pl
from jax.experimental.pallas import tpu as pltpu
from jax.experimental.pallas import tpu_sc as plsc

N = 10000
E = 160000
D = 128
DQK = 32
DATTR = 16
NB = 8
RMAX = 5.0
PPOW = 6.0
NLAYERS = 2

NC = 2
NS = 16
NW = NC * NS
NP = 10240
EP = 163840
ET = EP // NW
BATCH = 128
NBATCH = ET // BATCH
NGRP = ET // 16

_NEG_INF = float("-inf")


def _silu(x):
    return x * jax.nn.sigmoid(x)


def _k_pre_body(evt_ref, freq_ref, est_ref, eat_ref):
    ev = evt_ref[...]
    r2 = jnp.sum(ev * ev, axis=0, keepdims=True)
    r = jnp.sqrt(r2)
    freq = freq_ref[...]
    x = r / RMAX
    p = PPOW
    x2 = x * x
    x3 = x2 * x
    x6 = x3 * x3
    x7 = x6 * x
    x8 = x7 * x
    cut = (1.0
           - ((p + 1.0) * (p + 2.0) / 2.0) * x6
           + p * (p + 2.0) * x7
           - (p * (p + 1.0) / 2.0) * x8)
    cut = cut * (x < 1.0).astype(jnp.float32)
    est_ref[...] = (2.0 / RMAX) * jnp.sin(freq * r / RMAX) / r * cut
    u = ev / r
    s3 = math.sqrt(3.0)
    eat_ref[...] = jnp.concatenate(
        [jnp.ones_like(r), s3 * u[1:2], s3 * u[2:3], s3 * u[0:1]],
        axis=0)


def _edge_pre(evt, freq):
    BE = 2048
    grid = (EP // BE,)
    return pl.pallas_call(
        _k_pre_body,
        grid=grid,
        in_specs=[
            pl.BlockSpec((3, BE), lambda i: (0, i)),
            pl.BlockSpec((NB, 1), lambda i: (0, 0)),
        ],
        out_specs=[
            pl.BlockSpec((NB, BE), lambda i: (0, i)),
            pl.BlockSpec((4, BE), lambda i: (0, i)),
        ],
        out_shape=[
            jax.ShapeDtypeStruct((NB, EP), jnp.float32),
            jax.ShapeDtypeStruct((4, EP), jnp.float32),
        ],
    )(evt, freq.reshape(NB, 1))


def _k_mlp_body(est_ref, eat_ref, k1_ref, k2_ref, k3_ref, kk2_ref,
                v1_ref, v2_ref, v3_ref, vv2_ref, ck_ref, cv_ref):
    es = est_ref[...]
    ea = eat_ref[...]
    h = _silu(jnp.dot(k1_ref[...], es, preferred_element_type=jnp.float32))
    h = _silu(jnp.dot(k2_ref[...], h, preferred_element_type=jnp.float32))
    wkt = jnp.dot(k3_ref[...], h, preferred_element_type=jnp.float32)
    ckt = wkt * jnp.dot(kk2_ref[...], ea, preferred_element_type=jnp.float32)
    ck_ref[...] = jnp.transpose(ckt)
    hv = _silu(jnp.dot(v1_ref[...], es, preferred_element_type=jnp.float32))
    hv = _silu(jnp.dot(v2_ref[...], hv, preferred_element_type=jnp.float32))
    wvt = jnp.dot(v3_ref[...], hv, preferred_element_type=jnp.float32)
    cvt = wvt * jnp.dot(vv2_ref[...], ea, preferred_element_type=jnp.float32)
    cv_ref[...] = jnp.transpose(cvt)


def _edge_mlp(est, eat, fk1, fk2, fk3, wk2, fv1, fv2, fv3, wv2):
    BE = 2048
    grid = (EP // BE,)
    full = lambda shape: pl.BlockSpec(shape, lambda i: (0, 0))
    return pl.pallas_call(
        _k_mlp_body,
        grid=grid,
        in_specs=[
            pl.BlockSpec((NB, BE), lambda i: (0, i)),
            pl.BlockSpec((4, BE), lambda i: (0, i)),
            full((8, NB)), full((8, 8)), full((DQK, 8)), full((DQK, 4)),
            full((8, NB)), full((8, 8)), full((D, 8)), full((D, 4)),
        ],
        out_specs=[
            pl.BlockSpec((BE, DQK), lambda i: (i, 0)),
            pl.BlockSpec((BE, D), lambda i: (i, 0)),
        ],
        out_shape=[
            jax.ShapeDtypeStruct((EP, DQK), jnp.float32),
            jax.ShapeDtypeStruct((EP, D), jnp.float32),
        ],
    )(est, eat, fk1.T, fk2.T, fk3.T, wk2.T, fv1.T, fv2.T, fv3.T, wv2.T)


def _k_tables_body(x_ref, na_ref, wq_ref, wk1_ref, wv1_ref, ws1_ref, ws2_ref,
                   qk_ref, vt_ref, sc_ref):
    x = x_ref[...]
    q = jnp.dot(x, wq_ref[...], preferred_element_type=jnp.float32)
    k = jnp.dot(x, wk1_ref[...], preferred_element_type=jnp.float32)
    z = jnp.zeros_like(x[:, :64])
    qk_ref[...] = jnp.concatenate([q, k, z], axis=1)
    vt_ref[...] = jnp.dot(x, wv1_ref[...], preferred_element_type=jnp.float32)
    sc_ref[...] = (jnp.dot(x, ws1_ref[...], preferred_element_type=jnp.float32)
                   * jnp.dot(na_ref[...], ws2_ref[...],
                             preferred_element_type=jnp.float32))


def _tables(x_p, na_p, wq, wk1, wv1, ws1, ws2):
    BN = 1024
    grid = (NP // BN,)
    return pl.pallas_call(
        _k_tables_body,
        grid=grid,
        in_specs=[
            pl.BlockSpec((BN, D), lambda i: (i, 0)),
            pl.BlockSpec((BN, DATTR), lambda i: (i, 0)),
            pl.BlockSpec((D, DQK), lambda i: (0, 0)),
            pl.BlockSpec((D, DQK), lambda i: (0, 0)),
            pl.BlockSpec((D, D), lambda i: (0, 0)),
            pl.BlockSpec((D, D), lambda i: (0, 0)),
            pl.BlockSpec((DATTR, D), lambda i: (0, 0)),
        ],
        out_specs=[
            pl.BlockSpec((BN, D), lambda i: (i, 0)),
            pl.BlockSpec((BN, D), lambda i: (i, 0)),
            pl.BlockSpec((BN, D), lambda i: (i, 0)),
        ],
        out_shape=[
            jax.ShapeDtypeStruct((NP, D), jnp.float32),
            jax.ShapeDtypeStruct((NP, D), jnp.float32),
            jax.ShapeDtypeStruct((NP, D), jnp.float32),
        ],
    )(x_p, na_p, wq, wk1, wv1, ws1, ws2)


def _k_logit_body(qkg_ref, ck_ref, out_ref):
    qkg = qkg_ref[...]
    t = qkg[:, 0:DQK] * qkg[:, DQK:2 * DQK] * ck_ref[...]
    out_ref[...] = jnp.dot(
        t, jnp.full((DQK, 1), 1.0 / math.sqrt(DQK), jnp.float32),
        preferred_element_type=jnp.float32)


def _logits(qkg, ck):
    BE = 2048
    grid = (EP // BE,)
    return pl.pallas_call(
        _k_logit_body,
        grid=grid,
        in_specs=[
            pl.BlockSpec((BE, D), lambda i: (i, 0)),
            pl.BlockSpec((BE, DQK), lambda i: (i, 0)),
        ],
        out_specs=pl.BlockSpec((BE, 1), lambda i: (i, 0)),
        out_shape=jax.ShapeDtypeStruct((EP, 1), jnp.float32),
    )(qkg, ck)


def _k_reduce_body(part_ref, out_ref, *, op):
    t = jnp.transpose(part_ref[...])
    if op == "max":
        r = jnp.max(t, axis=1, keepdims=True)
        r = jnp.where(jnp.isfinite(r), r, 0.0)
    else:
        r = jnp.sum(t, axis=1, keepdims=True)
    out_ref[...] = r


def _reduce_parts(part, op):
    BN = 512
    grid = (NP // BN,)
    return pl.pallas_call(
        functools.partial(_k_reduce_body, op=op),
        grid=grid,
        in_specs=[pl.BlockSpec((NW, BN), lambda i: (0, i))],
        out_specs=pl.BlockSpec((BN, 1), lambda i: (i, 0)),
        out_shape=jax.ShapeDtypeStruct((NP, 1), jnp.float32),
    )(part)


def _k_wcv_body(cv_ref, ex_ref, out_ref):
    out_ref[...] = ex_ref[...] * cv_ref[...]


def _wcv(cv, ex):
    BE = 2048
    grid = (EP // BE,)
    return pl.pallas_call(
        _k_wcv_body,
        grid=grid,
        in_specs=[
            pl.BlockSpec((BE, D), lambda i: (i, 0)),
            pl.BlockSpec((BE, 1), lambda i: (i, 0)),
        ],
        out_specs=pl.BlockSpec((BE, D), lambda i: (i, 0)),
        out_shape=jax.ShapeDtypeStruct((EP, D), jnp.float32),
    )(cv, ex)


def _k_combine_body(agg_ref, den_ref, sc_ref, out_ref):
    a = agg_ref[0] + agg_ref[1]
    out_ref[...] = sc_ref[...] + a / (den_ref[...] + 1e-9)


def _combine(agg, den, sc):
    BN = 1024
    grid = (NP // BN,)
    return pl.pallas_call(
        _k_combine_body,
        grid=grid,
        in_specs=[
            pl.BlockSpec((2, BN, D), lambda i: (0, i, 0)),
            pl.BlockSpec((BN, 1), lambda i: (i, 0)),
            pl.BlockSpec((BN, D), lambda i: (i, 0)),
        ],
        out_specs=pl.BlockSpec((BN, D), lambda i: (i, 0)),
        out_shape=jax.ShapeDtypeStruct((NP, D), jnp.float32),
    )(agg, den, sc)


@functools.cache
def _mesh():
    return plsc.VectorSubcoreMesh(core_axis_name="c", subcore_axis_name="s",
                                  num_cores=NC, num_subcores=NS)


def _wid():
    return lax.axis_index("s") * NC + lax.axis_index("c")


NCHAIN = 4


def _seg_scan_rmw(keys, vals, kbuf, cbuf, arr, op):
    ident = _NEG_INF if op == "max" else 0.0
    sk, sv = plsc.sort_key_val(keys, vals)
    kbuf[pl.ds(16, 16)] = sk
    nxt = kbuf[pl.ds(17, 16)]
    last = sk != nxt
    c = sv
    for s in (1, 2, 4, 8):
        cbuf[pl.ds(16, 16)] = c
        ks = kbuf[pl.ds(16 - s, 16)]
        cs = cbuf[pl.ds(16 - s, 16)]
        eq = sk == ks
        contrib = jnp.where(eq, cs, jnp.full((16,), ident, jnp.float32))
        if op == "max":
            c = jnp.maximum(c, contrib)
        else:
            c = c + contrib
    cur = plsc.load_gather(arr, [sk], mask=last)
    if op == "max":
        new = jnp.maximum(cur, c)
    else:
        new = cur + c
    plsc.store_scatter(arr, [sk], new, mask=last)


def _fill(ref, n, value, dtype):
    v = jnp.full((16,), value, dtype)

    def body(i, _):
        ref[pl.ds(i * 16, 16)] = v
        return 0

    lax.fori_loop(0, n // 16, body, 0)


def _sc_gather_qk(qkt, dst_p, src_p):
    def body(qkt_hbm, dst_hbm, src_hbm, qkg_hbm,
             idx_d, idx_s, rows_q, rows_k, sem_q, sem_k):
        base = _wid() * ET

        def step(g, _):
            off = base + g * BATCH
            pltpu.sync_copy(dst_hbm.at[pl.ds(off, BATCH)], idx_d)
            pltpu.sync_copy(src_hbm.at[pl.ds(off, BATCH)], idx_s)
            cq = pltpu.async_copy(qkt_hbm.at[idx_d], rows_q, sem_q)
            ck = pltpu.async_copy(qkt_hbm.at[idx_s], rows_k, sem_k)
            cq.wait()
            ck.wait()

            def merge(r, _):
                rows_q[r, pl.ds(DQK, 16)] = rows_k[r, pl.ds(DQK, 16)]
                rows_q[r, pl.ds(DQK + 16, 16)] = rows_k[r, pl.ds(DQK + 16, 16)]
                return 0

            lax.fori_loop(0, BATCH, merge, 0)
            pltpu.sync_copy(rows_q, qkg_hbm.at[pl.ds(off, BATCH), :])
            return 0

        lax.fori_loop(0, NBATCH, step, 0)

    fn = pl.kernel(
        body,
        out_type=jax.ShapeDtypeStruct((EP, D), jnp.float32),
        mesh=_mesh(),
        compiler_params=pltpu.CompilerParams(needs_layout_passes=False),
        scratch_types=[
            pltpu.VMEM((BATCH,), jnp.int32),
            pltpu.VMEM((BATCH,), jnp.int32),
            pltpu.VMEM((BATCH, D), jnp.float32),
            pltpu.VMEM((BATCH, D), jnp.float32),
            pltpu.SemaphoreType.DMA,
            pltpu.SemaphoreType.DMA,
        ],
    )
    return fn(qkt, dst_p, src_p)


def _sc_segmax(logit, dst_p):
    def body(logit_hbm, dst_hbm, mpart_hbm,
             mv0, mv1, mv2, mv3, logit_v, dst_v,
             kb0, kb1, kb2, kb3, cb0, cb1, cb2, cb3):
        mvs = (mv0, mv1, mv2, mv3)
        kbs = (kb0, kb1, kb2, kb3)
        cbs = (cb0, cb1, cb2, cb3)
        w = _wid()
        base = w * ET
        pltpu.sync_copy(logit_hbm.at[pl.ds(base, ET)], logit_v)
        pltpu.sync_copy(dst_hbm.at[pl.ds(base, ET)], dst_v)
        ninf = jnp.full((16,), _NEG_INF, jnp.float32)

        def ifill(i, _):
            for c in range(NCHAIN):
                mvs[c][pl.ds(i * 16, 16)] = ninf
            return 0

        lax.fori_loop(0, NP // 16, ifill, 0)
        for c in range(NCHAIN):
            kbs[c][pl.ds(0, 16)] = jnp.full((16,), -1, jnp.int32)
            kbs[c][pl.ds(32, 16)] = jnp.full((16,), -2, jnp.int32)
            cbs[c][pl.ds(0, 16)] = ninf

        def step(go, _):
            for c in range(NCHAIN):
                off = go * (16 * NCHAIN) + c * 16
                keys = dst_v[pl.ds(off, 16)]
                vals = logit_v[pl.ds(off, 16)]
                _seg_scan_rmw(keys, vals, kbs[c], cbs[c], mvs[c], "max")
            return 0

        lax.fori_loop(0, NGRP // NCHAIN, step, 0)

        def mstep(i, _):
            sl = pl.ds(i * 16, 16)
            m01 = jnp.maximum(mv0[sl], mv1[sl])
            m23 = jnp.maximum(mv2[sl], mv3[sl])
            mv0[sl] = jnp.maximum(m01, m23)
            return 0

        lax.fori_loop(0, NP // 16, mstep, 0)
        pltpu.sync_copy(mv0, mpart_hbm.at[w])

    fn = pl.kernel(
        body,
        out_type=jax.ShapeDtypeStruct((NW, NP), jnp.float32),
        mesh=_mesh(),
        compiler_params=pltpu.CompilerParams(needs_layout_passes=False),
        scratch_types=(
            [pltpu.VMEM((NP,), jnp.float32)] * NCHAIN
            + [pltpu.VMEM((ET,), jnp.float32), pltpu.VMEM((ET,), jnp.int32)]
            + [pltpu.VMEM((48,), jnp.int32)] * NCHAIN
            + [pltpu.VMEM((48,), jnp.float32)] * NCHAIN
        ),
    )
    return fn(logit, dst_p)


def _sc_exden(logit, dst_p, m):
    def body(logit_hbm, dst_hbm, m_hbm, ex_hbm, dpart_hbm,
             mv, dv0, dv1, dv2, dv3, logit_v, dst_v, ex_v,
             kb0, kb1, kb2, kb3, cb0, cb1, cb2, cb3):
        dvs = (dv0, dv1, dv2, dv3)
        kbs = (kb0, kb1, kb2, kb3)
        cbs = (cb0, cb1, cb2, cb3)
        w = _wid()
        base = w * ET
        pltpu.sync_copy(logit_hbm.at[pl.ds(base, ET)], logit_v)
        pltpu.sync_copy(dst_hbm.at[pl.ds(base, ET)], dst_v)
        pltpu.sync_copy(m_hbm, mv)
        zero = jnp.zeros((16,), jnp.float32)

        def ifill(i, _):
            for c in range(NCHAIN):
                dvs[c][pl.ds(i * 16, 16)] = zero
            return 0

        lax.fori_loop(0, NP // 16, ifill, 0)
        for c in range(NCHAIN):
            kbs[c][pl.ds(0, 16)] = jnp.full((16,), -1, jnp.int32)
            kbs[c][pl.ds(32, 16)] = jnp.full((16,), -2, jnp.int32)
            cbs[c][pl.ds(0, 16)] = zero

        def step(go, _):
            for c in range(NCHAIN):
                off = go * (16 * NCHAIN) + c * 16
                keys = dst_v[pl.ds(off, 16)]
                lg = logit_v[pl.ds(off, 16)]
                md = plsc.load_gather(mv, [keys])
                ex = jnp.exp(lg - md)
                ex_v[pl.ds(off, 16)] = ex
                _seg_scan_rmw(keys, ex, kbs[c], cbs[c], dvs[c], "sum")
            return 0

        lax.fori_loop(0, NGRP // NCHAIN, step, 0)

        def mstep(i, _):
            sl = pl.ds(i * 16, 16)
            d01 = dv0[sl] + dv1[sl]
            d23 = dv2[sl] + dv3[sl]
            dv0[sl] = d01 + d23
            return 0

        lax.fori_loop(0, NP // 16, mstep, 0)
        pltpu.sync_copy(ex_v, ex_hbm.at[pl.ds(base, ET)])
        pltpu.sync_copy(dv0, dpart_hbm.at[w])

    fn = pl.kernel(
        body,
        out_type=(
            jax.ShapeDtypeStruct((EP,), jnp.float32),
            jax.ShapeDtypeStruct((NW, NP), jnp.float32),
        ),
        mesh=_mesh(),
        compiler_params=pltpu.CompilerParams(needs_layout_passes=False),
        scratch_types=(
            [pltpu.VMEM((NP,), jnp.float32)] * (NCHAIN + 1)
            + [pltpu.VMEM((ET,), jnp.float32), pltpu.VMEM((ET,), jnp.int32),
               pltpu.VMEM((ET,), jnp.float32)]
            + [pltpu.VMEM((48,), jnp.int32)] * NCHAIN
            + [pltpu.VMEM((48,), jnp.float32)] * NCHAIN
        ),
    )
    return fn(logit, dst_p, m)


def _sc_agg(vt, wcv, src_p, dst_p):
    ROWS_PER_TILE = NP // NS
    ZCH = 64

    def body(vt_hbm, wcv_hbm, src_hbm, dst_hbm, agg_hbm,
             idx_s, idx_d, vrows, wrows, zbuf, shared_agg, sem_v):
        c = lax.axis_index("c")
        s = lax.axis_index("s")
        base = _wid() * ET

        zv = jnp.zeros((16,), jnp.float32)

        def zfill(r, _):
            for cc in range(D // 16):
                zbuf[r, pl.ds(cc * 16, 16)] = zv
            return 0

        lax.fori_loop(0, ZCH, zfill, 0)
        r0 = s * ROWS_PER_TILE

        def zstep(i, _):
            pltpu.sync_copy(zbuf, shared_agg.at[pl.ds(r0 + i * ZCH, ZCH), :])
            return 0

        lax.fori_loop(0, ROWS_PER_TILE // ZCH, zstep, 0)
        plsc.subcore_barrier()

        def step(g, _):
            off = base + g * BATCH
            pltpu.sync_copy(src_hbm.at[pl.ds(off, BATCH)], idx_s)
            cv = pltpu.async_copy(vt_hbm.at[idx_s], vrows, sem_v)
            pltpu.sync_copy(wcv_hbm.at[pl.ds(off, BATCH), :], wrows)
            pltpu.sync_copy(dst_hbm.at[pl.ds(off, BATCH)], idx_d)
            cv.wait()

            def mul_row(r, _):
                for cc in range(D // 16):
                    sl = pl.ds(cc * 16, 16)
                    vrows[r, sl] = vrows[r, sl] * wrows[r, sl]
                return 0

            lax.fori_loop(0, BATCH, mul_row, 0)
            pltpu.sync_copy(vrows, shared_agg.at[idx_d], add=True)
            return 0

        lax.fori_loop(0, NBATCH, step, 0)
        plsc.subcore_barrier()
        pltpu.sync_copy(shared_agg.at[pl.ds(r0, ROWS_PER_TILE), :],
                        agg_hbm.at[c, pl.ds(r0, ROWS_PER_TILE), :])

    fn = pl.kernel(
        body,
        out_type=jax.ShapeDtypeStruct((NC, NP, D), jnp.float32),
        mesh=_mesh(),
        compiler_params=pltpu.CompilerParams(needs_layout_passes=False),
        scratch_types=[
            pltpu.VMEM((BATCH,), jnp.int32),
            pltpu.VMEM((BATCH,), jnp.int32),
            pltpu.VMEM((BATCH, D), jnp.float32),
            pltpu.VMEM((BATCH, D), jnp.float32),
            pltpu.VMEM((ZCH, D), jnp.float32),
            pltpu.VMEM_SHARED((NP, D), jnp.float32),
            pltpu.SemaphoreType.DMA,
        ],
    )
    return fn(vt, wcv, src_p, dst_p)


def kernel(node_feature, node_attr, edge_src, edge_dst, edge_vec, bessel_freq,
           Wq, Wk1, Wk2, Wv1, Wv2, fck_w1, fck_w2, fck_w3,
           fcv_w1, fcv_w2, fcv_w3, Wsc1, Wsc2):
    f32 = jnp.float32
    x = jnp.pad(node_feature, ((0, NP - N), (0, 0)))
    na = jnp.pad(node_attr, ((0, NP - N), (0, 0)))
    pad_e = EP - E
    ev_p = jnp.concatenate(
        [edge_vec,
         jnp.broadcast_to(jnp.array([1.0, 0.0, 0.0], f32), (pad_e, 3))])
    src_p = jnp.concatenate([edge_src, jnp.full((pad_e,), N, jnp.int32)])
    dst_p = jnp.concatenate([edge_dst, jnp.full((pad_e,), N, jnp.int32)])

    est, eat = _edge_pre(jnp.transpose(ev_p), bessel_freq)
    cks = []
    cvs = []
    for i in range(NLAYERS):
        ck_i, cv_i = _edge_mlp(est, eat, fck_w1[i], fck_w2[i], fck_w3[i],
                               Wk2[i], fcv_w1[i], fcv_w2[i], fcv_w3[i],
                               Wv2[i])
        cks.append(ck_i)
        cvs.append(cv_i)

    for i in range(NLAYERS):
        qkt, vt, sc = _tables(x, na, Wq[i], Wk1[i], Wv1[i], Wsc1[i],
                              Wsc2[i])
        qkg = _sc_gather_qk(qkt, dst_p, src_p)
        logit2d = _logits(qkg, cks[i])
        logit = logit2d.reshape(EP)
        mpart = _sc_segmax(logit, dst_p)
        m = _reduce_parts(mpart, "max")
        ex, dpart = _sc_exden(logit, dst_p, m.reshape(NP))
        den = _reduce_parts(dpart, "sum")
        wcv = _wcv(cvs[i], ex.reshape(EP, 1))
        agg = _sc_agg(vt, wcv, src_p, dst_p)
        x = _combine(agg, den, sc)

    return x[:N]
